# baseline (speedup 1.0000x reference)
import jax
import jax.numpy as jnp
from jax import lax
from jax.experimental import pallas as pl
from jax.experimental.pallas import tpu as pltpu
from jax.experimental.pallas import tpu_sc as plsc

BATCH = 16384
NC = 2
NS = 16
NW = NC * NS
BPW = BATCH // NW
CHUNK = 128
NCHUNK = BPW // CHUNK
WIDE = 128

MLP_BB = 2048


def _sc_gather_body(uid_hbm, gcell_hbm, utab_hbm, gtab_hbm,
                    uout_hbm, gout_hbm,
                    uidx_v, gidx_v, rows_v, sem):
    c = lax.axis_index("c")
    s = lax.axis_index("s")
    wid = s * NC + c
    r0 = wid * NCHUNK
    b0 = wid * BPW

    pltpu.sync_copy(uid_hbm.at[pl.ds(r0, NCHUNK), :], uidx_v)
    pltpu.sync_copy(gcell_hbm.at[pl.ds(r0, NCHUNK), :], gidx_v)

    for j in range(NCHUNK):
        for i in range(CHUNK // 16):
            sl = (j, pl.ds(i * 16, 16))
            uidx_v[sl] = uidx_v[sl] >> 2
            gidx_v[sl] = gidx_v[sl] >> 4

    copies = []
    for j in range(NCHUNK):
        copies.append(pltpu.async_copy(
            utab_hbm.at[uidx_v.at[j]],
            rows_v.at[pl.ds(j * CHUNK, CHUNK)], sem))
    for cp in copies:
        cp.wait()
    pltpu.sync_copy(rows_v, uout_hbm.at[pl.ds(b0, BPW)])

    copies = []
    for j in range(NCHUNK):
        copies.append(pltpu.async_copy(
            gtab_hbm.at[gidx_v.at[j]],
            rows_v.at[pl.ds(j * CHUNK, CHUNK)], sem))
    for cp in copies:
        cp.wait()
    pltpu.sync_copy(rows_v, gout_hbm.at[pl.ds(b0, BPW)])


def _sc_gather(uid2d, gcell2d, utab_wide, gtab_wide):
    mesh = plsc.VectorSubcoreMesh(
        core_axis_name="c", subcore_axis_name="s",
        num_cores=NC, num_subcores=NS)
    fn = pl.kernel(
        _sc_gather_body,
        out_type=(
            jax.ShapeDtypeStruct((BATCH, WIDE), jnp.float32),
            jax.ShapeDtypeStruct((BATCH, WIDE), jnp.float32),
        ),
        mesh=mesh,
        scratch_types=[
            pltpu.VMEM((NCHUNK, CHUNK), jnp.int32),
            pltpu.VMEM((NCHUNK, CHUNK), jnp.int32),
            pltpu.VMEM((BPW, WIDE), jnp.float32),
            pltpu.SemaphoreType.DMA,
        ],
        name="sc_user_geo_gather",
    )
    return fn(uid2d, gcell2d, utab_wide, gtab_wide)


def _mlp_body(uwide, gwide, uid, gcell, age, sched, intr,
              atab, stab, w0, b0, w1, b1, w2, b2, out):
    f32 = jnp.float32
    hi = jax.lax.Precision.HIGHEST
    dn = (((1,), (0,)), ((), ()))

    ids_u = uid[...]
    ids_g = gcell[...]

    lane = lax.broadcasted_iota(jnp.int32, (MLP_BB, WIDE), 1)
    umask = (lane >> 5 == (ids_u & 3)).astype(f32)
    gmask = (lane >> 3 == (ids_g & 15)).astype(f32)

    qi = lax.broadcasted_iota(jnp.int32, (WIDE, 32), 0)
    qj = lax.broadcasted_iota(jnp.int32, (WIDE, 32), 1)
    qu = ((qi & 31) == qj).astype(f32)
    gi = lax.broadcasted_iota(jnp.int32, (WIDE, 8), 0)
    gj = lax.broadcasted_iota(jnp.int32, (WIDE, 8), 1)
    qg = ((gi & 7) == gj).astype(f32)

    u = lax.dot_general(uwide[...] * umask, qu, dn, precision=hi)
    geo = lax.dot_general(gwide[...] * gmask, qg, dn, precision=hi)

    ids_a = age[...]
    ids_s = sched[...]
    iot = lax.broadcasted_iota(jnp.int32, (MLP_BB, 16), 1)
    aoh = (iot == ids_a).astype(f32)
    soh = (iot == ids_s).astype(f32)
    a_emb = lax.dot_general(aoh, atab[...], dn, precision=hi)
    s_emb = lax.dot_general(soh, stab[...], dn, precision=hi)

    x = jnp.concatenate([u, geo, a_emb, s_emb, intr[...]], axis=1)
    h = lax.dot_general(x, w0[...], dn, precision=hi) + b0[...]
    h = jnp.maximum(h, 0.0)
    h = lax.dot_general(h, w1[...], dn, precision=hi) + b1[...]
    h = jnp.maximum(h, 0.0)
    o = lax.dot_general(h, w2[...], dn, precision=hi) + b2[...]

    n2 = jnp.sum(o * o, axis=1, keepdims=True)
    out[...] = o * lax.rsqrt(jnp.maximum(n2, 1e-24))


def _mlp(uwide, gwide, uid2d, gc2d, age2d, sched2d, interest,
         atab16, stab16, W0, b0, W1, b1, W2, b2):
    nblk = BATCH // MLP_BB
    bspec = lambda r, cols: pl.BlockSpec((r, cols), lambda i: (i, 0))
    full = lambda shape: pl.BlockSpec(shape, lambda i: (0, 0))
    return pl.pallas_call(
        _mlp_body,
        grid=(nblk,),
        in_specs=[
            bspec(MLP_BB, WIDE),
            bspec(MLP_BB, WIDE),
            bspec(MLP_BB, 1),
            bspec(MLP_BB, 1),
            bspec(MLP_BB, 1),
            bspec(MLP_BB, 1),
            bspec(MLP_BB, 64),
            full((16, 4)),
            full((16, 4)),
            full((112, 256)),
            full((1, 256)),
            full((256, 128)),
            full((1, 128)),
            full((128, 64)),
            full((1, 64)),
        ],
        out_specs=bspec(MLP_BB, 64),
        out_shape=jax.ShapeDtypeStruct((BATCH, 64), jnp.float32),
        compiler_params=pltpu.CompilerParams(
            dimension_semantics=("arbitrary",)),
        name="user_tower_mlp",
    )(uwide, gwide, uid2d, gc2d, age2d, sched2d, interest,
      atab16, stab16, W0, b0, W1, b1, W2, b2)


def kernel(user_ids, geo_cells, age_buckets, schedule_types,
           interest_vectors, user_table, geo_table, age_table, sched_table,
           W0, b0, W1, b1, W2, b2):
    uid = user_ids.astype(jnp.int32)
    gc = geo_cells.astype(jnp.int32)
    ab = age_buckets.astype(jnp.int32)
    st = schedule_types.astype(jnp.int32)

    uid2d = uid.reshape(128, 128)
    gc2d = gc.reshape(128, 128)
    utab_wide = user_table.reshape(-1, WIDE)
    gtab_wide = geo_table.reshape(-1, WIDE)

    uwide, gwide = _sc_gather(uid2d, gc2d, utab_wide, gtab_wide)

    atab16 = jnp.zeros((16, 4), jnp.float32).at[:age_table.shape[0]].set(age_table)
    stab16 = jnp.zeros((16, 4), jnp.float32).at[:sched_table.shape[0]].set(sched_table)

    return _mlp(uwide, gwide,
                uid.reshape(BATCH, 1), gc.reshape(BATCH, 1),
                ab.reshape(BATCH, 1), st.reshape(BATCH, 1),
                interest_vectors,
                atab16, stab16,
                W0, b0.reshape(1, -1), W1, b1.reshape(1, -1),
                W2, b2.reshape(1, -1))


# speedup vs baseline: 1.3554x; 1.3554x over previous
import jax
import jax.numpy as jnp
from jax import lax
from jax.experimental import pallas as pl
from jax.experimental.pallas import tpu as pltpu
from jax.experimental.pallas import tpu_sc as plsc

BATCH = 16384
NC = 2
NS = 16
NW = NC * NS
BPW = BATCH // NW
L = 16
NVEC = BPW // L
CHUNK = 128
NCHUNK = BPW // CHUNK

USER_D = 32
GEO_D = 8
GEO_W = 16

MLP_BB = 2048


def _sc_user_body(uid_hbm, utab_hbm, uout_hbm, uidx_v, rows_v, sem):
    c = lax.axis_index("c")
    s = lax.axis_index("s")
    wid = s * NC + c
    b0 = wid * BPW

    pltpu.sync_copy(uid_hbm.at[pl.ds(b0, BPW)], uidx_v)

    lane = lax.iota(jnp.int32, L)
    zeros = jnp.zeros((L,), jnp.int32)

    def body(j, _):
        v = uidx_v[pl.ds(j * L, L)]
        for l in range(L):
            r = jnp.sum(jnp.where(lane == l, v, zeros))
            pltpu.async_copy(
                utab_hbm.at[pl.ds(r, 1), :],
                rows_v.at[pl.ds(j * L + l, 1), :], sem)

        @pl.when(j > 0)
        def _():
            pltpu.make_async_copy(
                utab_hbm.at[pl.ds(0, L), :],
                rows_v.at[pl.ds((j - 1) * L, L), :],
                sem).wait()
        return None

    lax.fori_loop(0, NVEC, body, None)
    pltpu.make_async_copy(
        utab_hbm.at[pl.ds(0, L), :],
        rows_v.at[pl.ds((NVEC - 1) * L, L), :],
        sem).wait()

    pltpu.sync_copy(rows_v, uout_hbm.at[pl.ds(b0, BPW)])


def _sc_user_gather(uid, user_table):
    mesh = plsc.VectorSubcoreMesh(
        core_axis_name="c", subcore_axis_name="s",
        num_cores=NC, num_subcores=NS)
    fn = pl.kernel(
        _sc_user_body,
        out_type=jax.ShapeDtypeStruct((BATCH, USER_D), jnp.float32),
        mesh=mesh,
        scratch_types=[
            pltpu.VMEM((BPW,), jnp.int32),
            pltpu.VMEM((BPW, USER_D), jnp.float32),
            pltpu.SemaphoreType.DMA,
        ],
        compiler_params=pltpu.CompilerParams(needs_layout_passes=False),
        name="sc_user_gather",
    )
    return fn(uid, user_table)


def _sc_geo_body(gcell_hbm, gtab_hbm, gout_hbm, gidx_v, grows_v, sem):
    c = lax.axis_index("c")
    s = lax.axis_index("s")
    wid = s * NC + c
    r0 = wid * NCHUNK
    b0 = wid * BPW

    pltpu.sync_copy(gcell_hbm.at[pl.ds(r0, NCHUNK), :], gidx_v)

    for j in range(NCHUNK):
        for i in range(CHUNK // L):
            sl = (j, pl.ds(i * L, L))
            gidx_v[sl] = gidx_v[sl] >> 1

    copies = []
    for j in range(NCHUNK):
        copies.append(pltpu.async_copy(
            gtab_hbm.at[gidx_v.at[j]],
            grows_v.at[pl.ds(j * CHUNK, CHUNK)], sem))
    for cp in copies:
        cp.wait()

    pltpu.sync_copy(grows_v, gout_hbm.at[pl.ds(b0, BPW)])


def _sc_geo_gather(gc2d, gtab16):
    mesh = plsc.VectorSubcoreMesh(
        core_axis_name="c", subcore_axis_name="s",
        num_cores=NC, num_subcores=NS)
    fn = pl.kernel(
        _sc_geo_body,
        out_type=jax.ShapeDtypeStruct((BATCH, GEO_W), jnp.float32),
        mesh=mesh,
        scratch_types=[
            pltpu.VMEM((NCHUNK, CHUNK), jnp.int32),
            pltpu.VMEM((BPW, GEO_W), jnp.float32),
            pltpu.SemaphoreType.DMA,
        ],
        compiler_params=pltpu.CompilerParams(use_tc_tiling_on_sc=False),
        name="sc_geo_gather",
    )
    return fn(gc2d, gtab16)


def _mlp_body(uemb, g16, gcell, age, sched, intr,
              atab, stab, w0, b0, w1, b1, w2, b2, out):
    f32 = jnp.float32
    hi = jax.lax.Precision.HIGHEST
    dn = (((1,), (0,)), ((), ()))

    u = uemb[...]
    g = g16[...]
    par = gcell[...]
    geo = jnp.where((par & 1) == 0, g[:, :GEO_D], g[:, GEO_D:])

    ids_a = age[...]
    ids_s = sched[...]
    iot = lax.broadcasted_iota(jnp.int32, (MLP_BB, 16), 1)
    aoh = (iot == ids_a).astype(f32)
    soh = (iot == ids_s).astype(f32)
    a_emb = lax.dot_general(aoh, atab[...], dn, precision=hi)
    s_emb = lax.dot_general(soh, stab[...], dn, precision=hi)

    x = jnp.concatenate([u, geo, a_emb, s_emb, intr[...]], axis=1)
    h = lax.dot_general(x, w0[...], dn, precision=hi) + b0[...]
    h = jnp.maximum(h, 0.0)
    h = lax.dot_general(h, w1[...], dn, precision=hi) + b1[...]
    h = jnp.maximum(h, 0.0)
    o = lax.dot_general(h, w2[...], dn, precision=hi) + b2[...]

    n2 = jnp.sum(o * o, axis=1, keepdims=True)
    out[...] = o * lax.rsqrt(jnp.maximum(n2, 1e-24))


def _mlp(uemb, g16, gc2d, age2d, sched2d, interest,
         atab16, stab16, W0, b0, W1, b1, W2, b2):
    nblk = BATCH // MLP_BB
    bspec = lambda r, cols: pl.BlockSpec((r, cols), lambda i: (i, 0))
    full = lambda shape: pl.BlockSpec(shape, lambda i: (0, 0))
    return pl.pallas_call(
        _mlp_body,
        grid=(nblk,),
        in_specs=[
            bspec(MLP_BB, USER_D),
            bspec(MLP_BB, GEO_W),
            bspec(MLP_BB, 1),
            bspec(MLP_BB, 1),
            bspec(MLP_BB, 1),
            bspec(MLP_BB, 64),
            full((16, 4)),
            full((16, 4)),
            full((112, 256)),
            full((1, 256)),
            full((256, 128)),
            full((1, 128)),
            full((128, 64)),
            full((1, 64)),
        ],
        out_specs=bspec(MLP_BB, 64),
        out_shape=jax.ShapeDtypeStruct((BATCH, 64), jnp.float32),
        compiler_params=pltpu.CompilerParams(
            dimension_semantics=("arbitrary",)),
        name="user_tower_mlp",
    )(uemb, g16, gc2d, age2d, sched2d, interest,
      atab16, stab16, W0, b0, W1, b1, W2, b2)


def kernel(user_ids, geo_cells, age_buckets, schedule_types,
           interest_vectors, user_table, geo_table, age_table, sched_table,
           W0, b0, W1, b1, W2, b2):
    uid = user_ids.astype(jnp.int32)
    gc = geo_cells.astype(jnp.int32)
    ab = age_buckets.astype(jnp.int32)
    st = schedule_types.astype(jnp.int32)

    uemb = _sc_user_gather(uid, user_table)
    g16 = _sc_geo_gather(gc.reshape(128, 128),
                         geo_table.reshape(-1, GEO_W))

    atab16 = jnp.zeros((16, 4), jnp.float32).at[:age_table.shape[0]].set(age_table)
    stab16 = jnp.zeros((16, 4), jnp.float32).at[:sched_table.shape[0]].set(sched_table)

    return _mlp(uemb, g16,
                gc.reshape(BATCH, 1), ab.reshape(BATCH, 1),
                st.reshape(BATCH, 1), interest_vectors,
                atab16, stab16,
                W0, b0.reshape(1, -1), W1, b1.reshape(1, -1),
                W2, b2.reshape(1, -1))


# speedup vs baseline: 1.3891x; 1.0249x over previous
import jax
import jax.numpy as jnp
from jax import lax
from jax.experimental import pallas as pl
from jax.experimental.pallas import tpu as pltpu
from jax.experimental.pallas import tpu_sc as plsc

BATCH = 16384
NC = 2
NS = 16
NW = NC * NS
BPW = BATCH // NW
L = 16
NVEC = BPW // L
CHUNK = 128
NCHUNK = BPW // CHUNK

USER_D = 32
GEO_D = 8

MLP_BB = 2048


def _sc_user_body(uid_hbm, utab_hbm, uout_hbm, uidx_v, rows_v, sem):
    c = lax.axis_index("c")
    s = lax.axis_index("s")
    wid = s * NC + c
    b0 = wid * BPW

    pltpu.sync_copy(uid_hbm.at[pl.ds(b0, BPW)], uidx_v)

    lane = lax.iota(jnp.int32, L)
    zeros = jnp.zeros((L,), jnp.int32)

    def body(j, _):
        v = uidx_v[pl.ds(j * L, L)]
        for l in range(L):
            r = jnp.sum(jnp.where(lane == l, v, zeros))
            pltpu.async_copy(
                utab_hbm.at[pl.ds(r, 1), :],
                rows_v.at[pl.ds(j * L + l, 1), :], sem)

        @pl.when(j > 0)
        def _():
            pltpu.make_async_copy(
                utab_hbm.at[pl.ds(0, L), :],
                rows_v.at[pl.ds((j - 1) * L, L), :],
                sem).wait()
        return None

    lax.fori_loop(0, NVEC, body, None)
    pltpu.make_async_copy(
        utab_hbm.at[pl.ds(0, L), :],
        rows_v.at[pl.ds((NVEC - 1) * L, L), :],
        sem).wait()

    pltpu.sync_copy(rows_v, uout_hbm.at[pl.ds(b0, BPW)])


def _sc_user_gather(uid, user_table):
    mesh = plsc.VectorSubcoreMesh(
        core_axis_name="c", subcore_axis_name="s",
        num_cores=NC, num_subcores=NS)
    fn = pl.kernel(
        _sc_user_body,
        out_type=jax.ShapeDtypeStruct((BATCH, USER_D), jnp.float32),
        mesh=mesh,
        scratch_types=[
            pltpu.VMEM((BPW,), jnp.int32),
            pltpu.VMEM((BPW, USER_D), jnp.float32),
            pltpu.SemaphoreType.DMA,
        ],
        compiler_params=pltpu.CompilerParams(needs_layout_passes=False),
        name="sc_user_gather",
    )
    return fn(uid, user_table)


def _sc_geo_body(gcell_hbm, gtab_hbm, gout_hbm, gidx_v, grows_v, sem):
    c = lax.axis_index("c")
    s = lax.axis_index("s")
    wid = s * NC + c
    r0 = wid * NCHUNK
    b0 = wid * BPW

    pltpu.sync_copy(gcell_hbm.at[pl.ds(r0, NCHUNK), :], gidx_v)

    copies = []
    for j in range(NCHUNK):
        copies.append(pltpu.async_copy(
            gtab_hbm.at[gidx_v.at[j]],
            grows_v.at[pl.ds(j * CHUNK, CHUNK)], sem))
    for cp in copies:
        cp.wait()

    pltpu.sync_copy(grows_v, gout_hbm.at[pl.ds(b0, BPW)])


def _sc_geo_gather(gc2d, gtab16):
    mesh = plsc.VectorSubcoreMesh(
        core_axis_name="c", subcore_axis_name="s",
        num_cores=NC, num_subcores=NS)
    fn = pl.kernel(
        _sc_geo_body,
        out_type=jax.ShapeDtypeStruct((BATCH, GEO_D), jnp.float32),
        mesh=mesh,
        scratch_types=[
            pltpu.VMEM((NCHUNK, CHUNK), jnp.int32),
            pltpu.VMEM((BPW, GEO_D), jnp.float32),
            pltpu.SemaphoreType.DMA,
        ],
        compiler_params=pltpu.CompilerParams(use_tc_tiling_on_sc=False),
        name="sc_geo_gather",
    )
    return fn(gc2d, gtab16)


def _mlp_body(uemb, gemb, age, sched, intr,
              atab, stab, w0, b0, w1, b1, w2, b2, out):
    f32 = jnp.float32
    hi = jax.lax.Precision.HIGHEST
    dn = (((1,), (0,)), ((), ()))

    u = uemb[...]
    geo = gemb[...]

    ids_a = age[...]
    ids_s = sched[...]
    iot = lax.broadcasted_iota(jnp.int32, (MLP_BB, 16), 1)
    aoh = (iot == ids_a).astype(f32)
    soh = (iot == ids_s).astype(f32)
    a_emb = lax.dot_general(aoh, atab[...], dn, precision=hi)
    s_emb = lax.dot_general(soh, stab[...], dn, precision=hi)

    x = jnp.concatenate([u, geo, a_emb, s_emb, intr[...]], axis=1)
    h = lax.dot_general(x, w0[...], dn, precision=hi) + b0[...]
    h = jnp.maximum(h, 0.0)
    h = lax.dot_general(h, w1[...], dn, precision=hi) + b1[...]
    h = jnp.maximum(h, 0.0)
    o = lax.dot_general(h, w2[...], dn, precision=hi) + b2[...]

    n2 = jnp.sum(o * o, axis=1, keepdims=True)
    out[...] = o * lax.rsqrt(jnp.maximum(n2, 1e-24))


def _mlp(uemb, gemb, age2d, sched2d, interest,
         atab16, stab16, W0, b0, W1, b1, W2, b2):
    nblk = BATCH // MLP_BB
    bspec = lambda r, cols: pl.BlockSpec((r, cols), lambda i: (i, 0))
    full = lambda shape: pl.BlockSpec(shape, lambda i: (0, 0))
    return pl.pallas_call(
        _mlp_body,
        grid=(nblk,),
        in_specs=[
            bspec(MLP_BB, USER_D),
            bspec(MLP_BB, GEO_D),
            bspec(MLP_BB, 1),
            bspec(MLP_BB, 1),
            bspec(MLP_BB, 64),
            full((16, 4)),
            full((16, 4)),
            full((112, 256)),
            full((1, 256)),
            full((256, 128)),
            full((1, 128)),
            full((128, 64)),
            full((1, 64)),
        ],
        out_specs=bspec(MLP_BB, 64),
        out_shape=jax.ShapeDtypeStruct((BATCH, 64), jnp.float32),
        compiler_params=pltpu.CompilerParams(
            dimension_semantics=("arbitrary",)),
        name="user_tower_mlp",
    )(uemb, gemb, age2d, sched2d, interest,
      atab16, stab16, W0, b0, W1, b1, W2, b2)


def kernel(user_ids, geo_cells, age_buckets, schedule_types,
           interest_vectors, user_table, geo_table, age_table, sched_table,
           W0, b0, W1, b1, W2, b2):
    uid = user_ids.astype(jnp.int32)
    gc = geo_cells.astype(jnp.int32)
    ab = age_buckets.astype(jnp.int32)
    st = schedule_types.astype(jnp.int32)

    uemb = _sc_user_gather(uid, user_table)
    gemb = _sc_geo_gather(gc.reshape(128, 128), geo_table)

    atab16 = jnp.zeros((16, 4), jnp.float32).at[:age_table.shape[0]].set(age_table)
    stab16 = jnp.zeros((16, 4), jnp.float32).at[:sched_table.shape[0]].set(sched_table)

    return _mlp(uemb, gemb,
                ab.reshape(BATCH, 1),
                st.reshape(BATCH, 1), interest_vectors,
                atab16, stab16,
                W0, b0.reshape(1, -1), W1, b1.reshape(1, -1),
                W2, b2.reshape(1, -1))
